# Initial kernel scaffold; baseline (speedup 1.0000x reference)
import jax
import jax.numpy as jnp
from jax.experimental import pallas as pl


def kernel(x, table):
    raise NotImplementedError("write your pallas kernel here")


# speedup vs baseline: 33.1633x; 33.1633x over previous
import functools

import jax
import jax.numpy as jnp
from jax import lax
from jax.experimental import pallas as pl
from jax.experimental.pallas import tpu as pltpu
from jax.experimental.pallas import tpu_sc as plsc

FIELD_NUM = 26
MULTI_LEN = 200
OFFSET = 900000
EMBED_DIM = 32
ONE_HOT = FIELD_NUM - 1


def _sum_body(x_ref, t_ref, o_ref):
    m = (x_ref[:, ONE_HOT:] != 0).astype(jnp.float32)
    o_ref[...] = jnp.dot(m, t_ref[...], preferred_element_type=jnp.float32)


def _multi_hot_sum(x, t200):
    batch, width = x.shape
    bb = 1024
    return pl.pallas_call(
        _sum_body,
        grid=(batch // bb,),
        in_specs=[
            pl.BlockSpec((bb, width), lambda i: (i, 0)),
            pl.BlockSpec((MULTI_LEN, EMBED_DIM), lambda i: (0, 0)),
        ],
        out_specs=pl.BlockSpec((bb, EMBED_DIM), lambda i: (i, 0)),
        out_shape=jax.ShapeDtypeStruct((batch, EMBED_DIM), jnp.float32),
    )(x, t200)


def _make_sc_gather(batch):
    info = plsc.get_sparse_core_info()
    nc, ns = info.num_cores, info.num_subcores
    nw = nc * ns
    bpw = batch // nw
    cb = 64
    cn = cb * ONE_HOT
    steps = bpw // cb

    mesh = plsc.VectorSubcoreMesh(core_axis_name="c", subcore_axis_name="s")

    @functools.partial(
        pl.kernel,
        mesh=mesh,
        compiler_params=pltpu.CompilerParams(use_tc_tiling_on_sc=False),
        out_type=jax.ShapeDtypeStruct((batch * FIELD_NUM, EMBED_DIM), jnp.float32),
        scratch_types=[
            pltpu.VMEM((cn,), jnp.int32),
            pltpu.VMEM((cn,), jnp.int32),
            pltpu.VMEM((cn, EMBED_DIM), jnp.float32),
            pltpu.VMEM((cb,), jnp.int32),
            pltpu.VMEM((cb, EMBED_DIM), jnp.float32),
            pltpu.SemaphoreType.DMA,
        ],
    )
    def k(table_hbm, sidx_hbm, didx_hbm, sum_hbm, sdidx_hbm, out_hbm,
          sidx_v, didx_v, rows_v, sdidx_v, sum_v, sem):
        wid = lax.axis_index("s") * nc + lax.axis_index("c")
        base = wid * bpw
        for step in range(steps):
            b0 = base + step * cb
            p0 = b0 * ONE_HOT
            pltpu.sync_copy(sidx_hbm.at[pl.ds(p0, cn)], sidx_v)
            pltpu.sync_copy(didx_hbm.at[pl.ds(p0, cn)], didx_v)
            pltpu.sync_copy(sdidx_hbm.at[pl.ds(b0, cb)], sdidx_v)
            pltpu.sync_copy(sum_hbm.at[pl.ds(b0, cb)], sum_v)
            pltpu.async_copy(table_hbm.at[sidx_v], rows_v, sem).wait()
            pltpu.async_copy(rows_v, out_hbm.at[didx_v], sem).wait()
            pltpu.async_copy(sum_v, out_hbm.at[sdidx_v], sem).wait()

    return k


def kernel(x, table):
    batch = x.shape[0]
    t200 = lax.slice(table, (OFFSET + 1, 0), (OFFSET + 1 + MULTI_LEN, EMBED_DIM))
    sum_embed = _multi_hot_sum(x, t200)
    src_idx = x[:, :ONE_HOT].reshape(-1)
    b_ar = jnp.arange(batch, dtype=jnp.int32)
    dst_idx = (b_ar[:, None] * FIELD_NUM
               + jnp.arange(ONE_HOT, dtype=jnp.int32)[None, :]).reshape(-1)
    sum_dst = b_ar * FIELD_NUM + ONE_HOT
    sc = _make_sc_gather(batch)
    out_flat = sc(table, src_idx, dst_idx, sum_embed, sum_dst)
    return out_flat.reshape(batch, FIELD_NUM, EMBED_DIM)
